# baseline (speedup 1.0000x reference)
import jax
import jax.numpy as jnp
from jax import lax
from jax.experimental import pallas as pl
from jax.experimental.pallas import tpu as pltpu
from jax.experimental.pallas import tpu_sc as plsc

EMBED_DIM = 32

_info = plsc.get_sparse_core_info()
_NC, _NS = _info.num_cores, _info.num_subcores
_NW = _NC * _NS
_LANES = _info.num_lanes

_TB = 5


def _make_gather(V: int, D: int, S: int, T: int):
    assert S % _NW == 0
    bw = S // _NW
    assert T % (2 * _TB) == 0
    n_chunks = T // _TB
    nb16 = bw // _LANES
    mesh = plsc.VectorSubcoreMesh(core_axis_name="c", subcore_axis_name="s")

    def body(ids_hbm, table_hbm, out_hbm, idx_v, rows_v, trans_v,
             sem_idx, sem_gat, sem_out):
        wid = lax.axis_index("s") * _NC + lax.axis_index("c")
        b0 = wid * bw
        iota = lax.iota(jnp.int32, _LANES)
        row_idx = [b16 * _LANES + iota for b16 in range(nb16)]

        def idx_start(i, buf):
            pltpu.make_async_copy(
                ids_hbm.at[pl.ds(i * _TB, _TB), pl.ds(b0, bw)],
                idx_v.at[buf], sem_idx).start()

        def idx_wait(buf):
            pltpu.make_async_copy(
                ids_hbm.at[pl.ds(0, _TB), pl.ds(b0, bw)],
                idx_v.at[buf], sem_idx).wait()

        def gat_start(buf):
            for tt in range(_TB):
                pltpu.make_async_copy(
                    table_hbm.at[idx_v.at[buf, tt]],
                    rows_v.at[buf, tt], sem_gat).start()

        def gat_wait(buf):
            for tt in range(_TB):
                pltpu.make_async_copy(
                    table_hbm.at[idx_v.at[buf, tt]],
                    rows_v.at[buf, tt], sem_gat).wait()

        def out_start(i, buf):
            pltpu.make_async_copy(
                trans_v.at[buf, slice(None), slice(None), pl.ds(0, bw)],
                out_hbm.at[pl.ds(i * _TB, _TB), slice(None), pl.ds(b0, bw)],
                sem_out).start()

        def out_wait(buf):
            pltpu.make_async_copy(
                trans_v.at[buf, slice(None), slice(None), pl.ds(0, bw)],
                out_hbm.at[pl.ds(0, _TB), slice(None), pl.ds(b0, bw)],
                sem_out).wait()

        def transpose(buf):
            d_idx = [dh * _LANES + iota for dh in range(D // _LANES)]
            for tt in range(_TB):
                rows2d = rows_v.at[buf, tt]
                trans2d = trans_v.at[buf, tt]

                def b_body(b, c, rows2d=rows2d, trans2d=trans2d):
                    bs = jnp.full((_LANES,), b, jnp.int32)
                    for dh in range(D // _LANES):
                        val = rows2d[b, pl.ds(dh * _LANES, _LANES)]
                        plsc.store_scatter(trans2d, [d_idx[dh], bs], val)
                    return c

                lax.fori_loop(0, bw, b_body, 0, unroll=8)

        def step(i, buf, has_next, has_next2, has_prev2):
            gat_wait(buf)
            if has_next:
                idx_wait(1 - buf)
                gat_start(1 - buf)
            if has_next2:
                idx_start(i + 2, buf)
            if has_prev2:
                out_wait(buf)
            transpose(buf)
            out_start(i, buf)

        idx_start(0, 0)
        idx_start(1, 1)
        idx_wait(0)
        gat_start(0)
        step(0, 0, True, True, False)
        step(1, 1, True, True, False)

        def pair(g, c):
            step(2 * g, 0, True, True, True)
            step(2 * g + 1, 1, True, True, True)
            return c

        lax.fori_loop(1, n_chunks // 2 - 1, pair, 0, unroll=False)

        step(n_chunks - 2, 0, True, False, True)
        step(n_chunks - 1, 1, False, False, True)
        out_wait(0)
        out_wait(1)

    return pl.kernel(
        body,
        out_type=jax.ShapeDtypeStruct((T, D, S), jnp.float32),
        mesh=mesh,
        scratch_types=[
            pltpu.VMEM((2, _TB, S // _NW), jnp.int32),
            pltpu.VMEM((2, _TB, S // _NW, D), jnp.float32),
            pltpu.VMEM((2, _TB, D, S // _NW + 1), jnp.float32),
            pltpu.SemaphoreType.DMA,
            pltpu.SemaphoreType.DMA,
            pltpu.SemaphoreType.DMA,
        ],
        compiler_params=pltpu.CompilerParams(
            use_tc_tiling_on_sc=False, needs_layout_passes=False),
    )


def kernel(input_ids, table):
    Bt, T = input_ids.shape
    V, D = table.shape
    ids_t = input_ids.T.astype(jnp.int32)
    out = _make_gather(V, D, Bt, T)(ids_t, table)
    return jnp.transpose(out, (2, 0, 1))


# speedup vs baseline: 1.1403x; 1.1403x over previous
import jax
import jax.numpy as jnp
from jax import lax
from jax.experimental import pallas as pl
from jax.experimental.pallas import tpu as pltpu
from jax.experimental.pallas import tpu_sc as plsc

EMBED_DIM = 32

_info = plsc.get_sparse_core_info()
_NC, _NS = _info.num_cores, _info.num_subcores
_NW = _NC * _NS
_LANES = _info.num_lanes

_TB = 5


def _make_gather(V: int, D: int, S: int, T: int):
    assert S % _NW == 0
    bw = S // _NW
    assert T % (2 * _TB) == 0
    n_chunks = T // _TB
    nb16 = bw // _LANES
    mesh = plsc.VectorSubcoreMesh(core_axis_name="c", subcore_axis_name="s")

    def body(ids_hbm, table_hbm, out_hbm, idx_v, rows_v, trans_v,
             sem_idx, sem_gat, sem_out):
        wid = lax.axis_index("s") * _NC + lax.axis_index("c")
        b0 = wid * bw
        iota = lax.iota(jnp.int32, _LANES)
        row_idx = [b16 * _LANES + iota for b16 in range(nb16)]

        def idx_start(i, buf):
            pltpu.make_async_copy(
                ids_hbm.at[pl.ds(i * _TB, _TB), pl.ds(b0, bw)],
                idx_v.at[buf], sem_idx).start()

        def idx_wait(buf):
            pltpu.make_async_copy(
                ids_hbm.at[pl.ds(0, _TB), pl.ds(b0, bw)],
                idx_v.at[buf], sem_idx).wait()

        def gat_start(buf):
            for tt in range(_TB):
                pltpu.make_async_copy(
                    table_hbm.at[idx_v.at[buf, tt]],
                    rows_v.at[buf, tt], sem_gat).start()

        def gat_wait(buf):
            for tt in range(_TB):
                pltpu.make_async_copy(
                    table_hbm.at[idx_v.at[buf, tt]],
                    rows_v.at[buf, tt], sem_gat).wait()

        def out_start(i, buf):
            for tt in range(_TB):
                pltpu.make_async_copy(
                    trans_v.at[buf, tt, slice(None), slice(None), pl.ds(0, bw)],
                    out_hbm.at[i * _TB + tt, slice(None), wid],
                    sem_out).start()

        def out_wait(buf):
            for tt in range(_TB):
                pltpu.make_async_copy(
                    trans_v.at[buf, tt, slice(None), slice(None), pl.ds(0, bw)],
                    out_hbm.at[tt, slice(None), wid],
                    sem_out).wait()

        def transpose(buf):
            d_vec = [dh * _LANES + iota for dh in range(D // _LANES)]
            rt_idx = [dv // 8 for dv in d_vec]
            dr_idx = [dv % 8 for dv in d_vec]
            for tt in range(_TB):
                rows2d = rows_v.at[buf, tt]
                trans3d = trans_v.at[buf, tt]

                def b_body(b, c, rows2d=rows2d, trans3d=trans3d):
                    bs = jnp.full((_LANES,), b, jnp.int32)
                    for dh in range(D // _LANES):
                        val = rows2d[b, pl.ds(dh * _LANES, _LANES)]
                        plsc.store_scatter(
                            trans3d, [rt_idx[dh], dr_idx[dh], bs], val)
                    return c

                lax.fori_loop(0, bw, b_body, 0, unroll=8)

        def step(i, buf, has_next, has_next2, has_prev2):
            gat_wait(buf)
            if has_next:
                idx_wait(1 - buf)
                gat_start(1 - buf)
            if has_next2:
                idx_start(i + 2, buf)
            if has_prev2:
                out_wait(buf)
            transpose(buf)
            out_start(i, buf)

        idx_start(0, 0)
        idx_start(1, 1)
        idx_wait(0)
        gat_start(0)
        step(0, 0, True, True, False)
        step(1, 1, True, True, False)

        def pair(g, c):
            step(2 * g, 0, True, True, True)
            step(2 * g + 1, 1, True, True, True)
            return c

        lax.fori_loop(1, n_chunks // 2 - 1, pair, 0, unroll=False)

        step(n_chunks - 2, 0, True, False, True)
        step(n_chunks - 1, 1, False, False, True)
        out_wait(0)
        out_wait(1)

    return pl.kernel(
        body,
        out_type=jax.ShapeDtypeStruct((T, D // 8, _NW, 8, S // _NW), jnp.float32),
        mesh=mesh,
        scratch_types=[
            pltpu.VMEM((2, _TB, S // _NW), jnp.int32),
            pltpu.VMEM((2, _TB, S // _NW, D), jnp.float32),
            pltpu.VMEM((2, _TB, D // 8, 8, S // _NW + 1), jnp.float32),
            pltpu.SemaphoreType.DMA,
            pltpu.SemaphoreType.DMA,
            pltpu.SemaphoreType.DMA,
        ],
        compiler_params=pltpu.CompilerParams(
            use_tc_tiling_on_sc=False, needs_layout_passes=False),
    )


def kernel(input_ids, table):
    Bt, T = input_ids.shape
    V, D = table.shape
    ids_t = input_ids.T.astype(jnp.int32)
    out5 = _make_gather(V, D, Bt, T)(ids_t, table)
    return out5.transpose(2, 4, 0, 1, 3).reshape(Bt, T, D)
